# Initial kernel scaffold; baseline (speedup 1.0000x reference)
import jax
import jax.numpy as jnp
from jax.experimental import pallas as pl


def kernel(x, embedding):
    raise NotImplementedError("write your pallas kernel here")


# speedup vs baseline: 1.2616x; 1.2616x over previous
import functools

import jax
import jax.numpy as jnp
from jax import lax
from jax.experimental import pallas as pl
from jax.experimental.pallas import tpu as pltpu, tpu_sc as plsc

_INFO = plsc.get_sparse_core_info()
_NC, _NS = _INFO.num_cores, _INFO.num_subcores
_NW = _NC * _NS

_CHUNK = 1024


@functools.partial(jax.jit, static_argnames=("n_rows", "n_cols"))
def _embed_lookup(x_flat, embedding, *, n_rows, n_cols):
    b = x_flat.shape[0]
    b_per_w = b // _NW
    n_chunks = b_per_w // _CHUNK
    d = embedding.shape[1]

    mesh = plsc.VectorSubcoreMesh(core_axis_name="c", subcore_axis_name="s")

    @functools.partial(
        pl.kernel,
        mesh=mesh,
        out_type=jax.ShapeDtypeStruct((b, d), jnp.float32),
        scratch_types=[
            pltpu.VMEM((_CHUNK,), jnp.int32),
            pltpu.VMEM((_CHUNK, d), jnp.float32),
            pltpu.SemaphoreType.DMA,
        ],
        compiler_params=pltpu.CompilerParams(use_tc_tiling_on_sc=False),
    )
    def body(table_hbm, idx_hbm, out_hbm, idx_v, rows_v, sem):
        wid = lax.axis_index("s") * _NC + lax.axis_index("c")
        base = wid * b_per_w
        for c in range(n_chunks):
            off = base + c * _CHUNK
            pltpu.sync_copy(idx_hbm.at[pl.ds(off, _CHUNK)], idx_v)
            pltpu.async_copy(table_hbm.at[idx_v], rows_v, sem).wait()
            pltpu.sync_copy(rows_v, out_hbm.at[pl.ds(off, _CHUNK)])

    out = body(embedding, x_flat)
    return out.reshape(n_rows, n_cols, d)


def kernel(x, embedding):
    if x.dtype != jnp.int32:
        x = jnp.round(x).astype(jnp.int32)
    n_rows, n_cols = x.shape
    return _embed_lookup(x.reshape(-1), embedding, n_rows=n_rows, n_cols=n_cols)


# speedup vs baseline: 1.2781x; 1.0131x over previous
import functools

import jax
import jax.numpy as jnp
from jax import lax
from jax.experimental import pallas as pl
from jax.experimental.pallas import tpu as pltpu, tpu_sc as plsc

_INFO = plsc.get_sparse_core_info()
_NC, _NS = _INFO.num_cores, _INFO.num_subcores
_NW = _NC * _NS

_CHUNK = 1280


@functools.partial(jax.jit, static_argnames=("n_rows", "n_cols"))
def _embed_lookup(x_flat, embedding, *, n_rows, n_cols):
    b = x_flat.shape[0]
    b_per_w = b // _NW
    n_chunks = b_per_w // _CHUNK
    d = embedding.shape[1]

    mesh = plsc.VectorSubcoreMesh(core_axis_name="c", subcore_axis_name="s")

    @functools.partial(
        pl.kernel,
        mesh=mesh,
        out_type=jax.ShapeDtypeStruct((b, d), jnp.float32),
        scratch_types=[
            pltpu.VMEM((b_per_w,), jnp.int32),
            pltpu.VMEM((_CHUNK, d), jnp.float32),
            pltpu.VMEM((_CHUNK, d), jnp.float32),
            pltpu.SemaphoreType.DMA,
            pltpu.SemaphoreType.DMA,
            pltpu.SemaphoreType.DMA,
            pltpu.SemaphoreType.DMA,
        ],
        compiler_params=pltpu.CompilerParams(use_tc_tiling_on_sc=False),
    )
    def body(table_hbm, idx_hbm, out_hbm, idx_v, rows_v0, rows_v1,
             gsem0, gsem1, wsem0, wsem1):
        wid = lax.axis_index("s") * _NC + lax.axis_index("c")
        base = wid * b_per_w
        pltpu.sync_copy(idx_hbm.at[pl.ds(base, b_per_w)], idx_v)
        rows = (rows_v0, rows_v1)
        gsems = (gsem0, gsem1)
        wsems = (wsem0, wsem1)
        gathers = [None, None]
        writes = [None, None]
        for c in range(n_chunks):
            ib = c % 2
            if writes[ib] is not None:
                writes[ib].wait()
            gathers[ib] = pltpu.async_copy(
                table_hbm.at[idx_v.at[pl.ds(c * _CHUNK, _CHUNK)]],
                rows[ib], gsems[ib])
            pb = 1 - ib
            if gathers[pb] is not None:
                gathers[pb].wait()
                writes[pb] = pltpu.async_copy(
                    rows[pb],
                    out_hbm.at[pl.ds(base + (c - 1) * _CHUNK, _CHUNK)],
                    wsems[pb])
        last = (n_chunks - 1) % 2
        gathers[last].wait()
        writes[last] = pltpu.async_copy(
            rows[last],
            out_hbm.at[pl.ds(base + (n_chunks - 1) * _CHUNK, _CHUNK)],
            wsems[last])
        writes[0].wait()
        writes[1].wait()

    out = body(embedding, x_flat)
    return out.reshape(n_rows, n_cols, d)


def kernel(x, embedding):
    if x.dtype != jnp.int32:
        x = jnp.round(x).astype(jnp.int32)
    n_rows, n_cols = x.shape
    return _embed_lookup(x.reshape(-1), embedding, n_rows=n_rows, n_cols=n_cols)


# speedup vs baseline: 1.2788x; 1.0005x over previous
import functools

import jax
import jax.numpy as jnp
from jax import lax
from jax.experimental import pallas as pl
from jax.experimental.pallas import tpu as pltpu, tpu_sc as plsc

_INFO = plsc.get_sparse_core_info()
_NC, _NS = _INFO.num_cores, _INFO.num_subcores
_NW = _NC * _NS

_CHUNK = 640
_NBUF = 4


@functools.partial(jax.jit, static_argnames=("n_rows", "n_cols"))
def _embed_lookup(x_flat, embedding, *, n_rows, n_cols):
    b = x_flat.shape[0]
    b_per_w = b // _NW
    n_chunks = b_per_w // _CHUNK
    d = embedding.shape[1]

    mesh = plsc.VectorSubcoreMesh(core_axis_name="c", subcore_axis_name="s")

    @functools.partial(
        pl.kernel,
        mesh=mesh,
        out_type=jax.ShapeDtypeStruct((b, d), jnp.float32),
        scratch_types=(
            [pltpu.VMEM((b_per_w,), jnp.int32)]
            + [pltpu.VMEM((_CHUNK, d), jnp.float32) for _ in range(_NBUF)]
            + [pltpu.SemaphoreType.DMA for _ in range(2 * _NBUF)]
        ),
        compiler_params=pltpu.CompilerParams(use_tc_tiling_on_sc=False),
    )
    def body(table_hbm, idx_hbm, out_hbm, idx_v, *bufs_and_sems):
        rows = bufs_and_sems[:_NBUF]
        gsems = bufs_and_sems[_NBUF:2 * _NBUF]
        wsems = bufs_and_sems[2 * _NBUF:]
        wid = lax.axis_index("s") * _NC + lax.axis_index("c")
        base = wid * b_per_w
        pltpu.sync_copy(idx_hbm.at[pl.ds(base, b_per_w)], idx_v)
        gathers = [None] * _NBUF
        writes = [None] * _NBUF
        for c in range(n_chunks + _NBUF - 1):
            if c < n_chunks:
                ib = c % _NBUF
                if writes[ib] is not None:
                    writes[ib].wait()
                gathers[ib] = pltpu.async_copy(
                    table_hbm.at[idx_v.at[pl.ds(c * _CHUNK, _CHUNK)]],
                    rows[ib], gsems[ib])
            cd = c - (_NBUF - 1)
            if cd >= 0:
                db = cd % _NBUF
                gathers[db].wait()
                writes[db] = pltpu.async_copy(
                    rows[db],
                    out_hbm.at[pl.ds(base + cd * _CHUNK, _CHUNK)],
                    wsems[db])
        for w in writes:
            w.wait()

    out = body(embedding, x_flat)
    return out.reshape(n_rows, n_cols, d)


def kernel(x, embedding):
    if x.dtype != jnp.int32:
        x = jnp.round(x).astype(jnp.int32)
    n_rows, n_cols = x.shape
    return _embed_lookup(x.reshape(-1), embedding, n_rows=n_rows, n_cols=n_cols)
